# baseline (speedup 1.0000x reference)
import functools

import jax
import jax.numpy as jnp
from jax import lax
from jax.experimental import pallas as pl
from jax.experimental.pallas import tpu as pltpu
from jax.experimental.pallas import tpu_sc as plsc

N_FIELDS = 26
VOCAB = 100000
EMB = 32
BATCH = 16384

_NW = 32
_ROWS_W = BATCH * N_FIELDS // _NW
_G = 128
_NG = _ROWS_W // _G


def _body(tab, xf, out, xv, idxg, rows, sem):
    wid = lax.axis_index("s") * 2 + lax.axis_index("c")
    base = wid * _ROWS_W
    pltpu.sync_copy(xf.at[pl.ds(base, _ROWS_W)], xv)

    lane = lax.iota(jnp.int32, 16)

    def step(g, _):
        for s in range(_G // 16):
            j0 = g * _G + s * 16
            pos = j0 + lane
            off = (pos % N_FIELDS) * VOCAB
            idxg[pl.ds(s * 16, 16)] = xv[pl.ds(j0, 16)] + off
        pltpu.async_copy(tab.at[idxg], rows, sem).wait()
        pltpu.sync_copy(rows, out.at[pl.ds(base + g * _G, _G)])
        return ()

    lax.fori_loop(0, _NG, step, ())


@jax.jit
def kernel(x, tables):
    tab = tables.reshape(N_FIELDS * VOCAB, EMB)
    xf = x.reshape(BATCH * N_FIELDS)
    mesh = plsc.VectorSubcoreMesh(core_axis_name="c", subcore_axis_name="s")
    run = functools.partial(
        pl.kernel,
        mesh=mesh,
        compiler_params=pltpu.CompilerParams(use_tc_tiling_on_sc=False),
        out_type=jax.ShapeDtypeStruct((BATCH * N_FIELDS, EMB), jnp.float32),
        scratch_types=[
            pltpu.VMEM((_ROWS_W,), jnp.int32),
            pltpu.VMEM((_G,), jnp.int32),
            pltpu.VMEM((_G, EMB), jnp.float32),
            pltpu.SemaphoreType.DMA,
        ],
    )(_body)
    out = run(tab, xf)
    return out.reshape(BATCH, N_FIELDS * EMB)
